# baseline (speedup 1.0000x reference)
import numpy as np
import jax
import jax.numpy as jnp
from jax.experimental import pallas as pl
from jax.experimental.pallas import tpu as pltpu

_N_ROWS = 16384
_N_COLS = 2048
_N_CHUNKS = 16
_CHUNK_R = _N_ROWS // _N_CHUNKS


def _uniform_consts() -> np.ndarray:
    ks0, ks1 = np.uint32(0), np.uint32(42)
    ks2 = np.uint32(ks0 ^ ks1 ^ np.uint32(0x1BD11BDA))
    ks = [ks0, ks1, ks2]
    rot = [(13, 15, 26, 6), (17, 29, 16, 24)]

    def rotl(x, r):
        r = np.uint32(r)
        return ((x << r) | (x >> np.uint32(32 - r))).astype(np.uint32)

    x0 = np.full(_N_COLS, ks0, dtype=np.uint32)
    x1 = (np.arange(_N_COLS, dtype=np.uint32) + ks1).astype(np.uint32)
    for i in range(5):
        for r in rot[i % 2]:
            x0 = (x0 + x1).astype(np.uint32)
            x1 = rotl(x1, r)
            x1 = (x1 ^ x0).astype(np.uint32)
        x0 = (x0 + ks[(i + 1) % 3]).astype(np.uint32)
        x1 = (x1 + ks[(i + 2) % 3] + np.uint32(i + 1)).astype(np.uint32)
    bits = (x0 ^ x1).astype(np.uint32)
    fb = ((bits >> np.uint32(9)) | np.uint32(0x3F800000)).astype(np.uint32)
    u = fb.view(np.float32) - np.float32(1.0)
    return np.broadcast_to(u, (8, _N_COLS)).copy()


_U_TILE = _uniform_consts()


def _dma_body(idx_ref, prob_ref, u_ref, in_hbm, out_hbm, row_vmem,
              bulk_sem, row_sem):
    for c in range(_N_CHUNKS):
        pltpu.make_async_copy(
            in_hbm.at[pl.ds(c * _CHUNK_R, _CHUNK_R), :],
            out_hbm.at[pl.ds(c * _CHUNK_R, _CHUNK_R), :],
            bulk_sem,
        ).start()

    idx = idx_ref[0]
    fetch = pltpu.make_async_copy(
        in_hbm.at[pl.ds(idx, 1), :], row_vmem.at[pl.ds(0, 1), :], row_sem)
    fetch.start()
    fetch.wait()
    sample = (u_ref[pl.ds(0, 1), :] < prob_ref[0]).astype(jnp.float32)
    row_vmem[pl.ds(0, 1), :] = jnp.abs(row_vmem[pl.ds(0, 1), :] - sample)

    for c in range(_N_CHUNKS):
        pltpu.make_async_copy(
            in_hbm.at[pl.ds(c * _CHUNK_R, _CHUNK_R), :],
            out_hbm.at[pl.ds(c * _CHUNK_R, _CHUNK_R), :],
            bulk_sem,
        ).wait()

    store = pltpu.make_async_copy(
        row_vmem.at[pl.ds(0, 1), :], out_hbm.at[pl.ds(idx, 1), :], row_sem)
    store.start()
    store.wait()


def kernel(tensor, tensor_slice_index, probability):
    idx = jnp.asarray(tensor_slice_index, jnp.int32).reshape((1,))
    prob = jnp.asarray(probability, jnp.float32).reshape((1,))
    u = jnp.asarray(_U_TILE)
    out = pl.pallas_call(
        _dma_body,
        in_specs=[
            pl.BlockSpec(memory_space=pltpu.SMEM),
            pl.BlockSpec(memory_space=pltpu.SMEM),
            pl.BlockSpec(memory_space=pltpu.VMEM),
            pl.BlockSpec(memory_space=pltpu.MemorySpace.HBM),
        ],
        out_specs=pl.BlockSpec(memory_space=pltpu.MemorySpace.HBM),
        out_shape=jax.ShapeDtypeStruct((_N_ROWS, _N_COLS), jnp.float32),
        scratch_shapes=[
            pltpu.VMEM((8, _N_COLS), jnp.float32),
            pltpu.SemaphoreType.DMA,
            pltpu.SemaphoreType.DMA,
        ],
    )(idx, prob, u, tensor)
    return (out, tensor_slice_index)


# speedup vs baseline: 37.8999x; 37.8999x over previous
import functools

import numpy as np
import jax
import jax.numpy as jnp
from jax import lax
from jax.experimental import pallas as pl
from jax.experimental.pallas import tpu as pltpu
from jax.experimental.pallas import tpu_sc as plsc

_N_ROWS = 16384
_N_COLS = 2048
_BLOCK_R = 1024
_N_WORKERS = 32
_COLS_PER_W = _N_COLS // _N_WORKERS
_LANES = 16


def _uniform_consts() -> np.ndarray:
    ks0, ks1 = np.uint32(0), np.uint32(42)
    ks2 = np.uint32(ks0 ^ ks1 ^ np.uint32(0x1BD11BDA))
    ks = [ks0, ks1, ks2]
    rot = [(13, 15, 26, 6), (17, 29, 16, 24)]

    def rotl(x, r):
        r = np.uint32(r)
        return ((x << r) | (x >> np.uint32(32 - r))).astype(np.uint32)

    x0 = np.full(_N_COLS, ks0, dtype=np.uint32)
    x1 = (np.arange(_N_COLS, dtype=np.uint32) + ks1).astype(np.uint32)
    for i in range(5):
        for r in rot[i % 2]:
            x0 = (x0 + x1).astype(np.uint32)
            x1 = rotl(x1, r)
            x1 = (x1 ^ x0).astype(np.uint32)
        x0 = (x0 + ks[(i + 1) % 3]).astype(np.uint32)
        x1 = (x1 + ks[(i + 2) % 3] + np.uint32(i + 1)).astype(np.uint32)
    bits = (x0 ^ x1).astype(np.uint32)
    fb = ((bits >> np.uint32(9)) | np.uint32(0x3F800000)).astype(np.uint32)
    u = fb.view(np.float32) - np.float32(1.0)
    return u


_U_ROW = _uniform_consts()


def _sc_flip_body(idx_hbm, prob_hbm, u_hbm, tensor_hbm, out_hbm,
                  idx_v, row_v, u_v, prob_v, res_v, sem):
    w = lax.axis_index("s") * 2 + lax.axis_index("c")
    base = w * _COLS_PER_W
    pltpu.sync_copy(idx_hbm, idx_v)
    pltpu.async_copy(tensor_hbm.at[idx_v], row_v, sem).wait()
    pltpu.sync_copy(u_hbm.at[pl.ds(base, _COLS_PER_W)], u_v)
    pltpu.sync_copy(prob_hbm, prob_v)
    pv = prob_v[...]
    one = jnp.full((_LANES,), 1.0, jnp.float32)
    zero = jnp.full((_LANES,), 0.0, jnp.float32)
    for j in range(_COLS_PER_W // _LANES):
        u16 = u_v[pl.ds(j * _LANES, _LANES)]
        r16 = row_v[0, pl.ds(base + j * _LANES, _LANES)]
        s16 = jnp.where(u16 < pv, one, zero)
        res_v[pl.ds(j * _LANES, _LANES)] = jnp.abs(r16 - s16)
    pltpu.sync_copy(res_v, out_hbm.at[pl.ds(base, _COLS_PER_W)])


def _sc_flip_row(idx, prob16, u, tensor):
    mesh = plsc.VectorSubcoreMesh(core_axis_name="c", subcore_axis_name="s")
    run = functools.partial(
        pl.kernel,
        out_type=jax.ShapeDtypeStruct((_N_COLS,), jnp.float32),
        mesh=mesh,
        scratch_types=[
            pltpu.VMEM((1,), jnp.int32),
            pltpu.VMEM((1, _N_COLS), jnp.float32),
            pltpu.VMEM((_COLS_PER_W,), jnp.float32),
            pltpu.VMEM((_LANES,), jnp.float32),
            pltpu.VMEM((_COLS_PER_W,), jnp.float32),
            pltpu.SemaphoreType.DMA,
        ],
    )(_sc_flip_body)
    return run(idx, prob16, u, tensor)


def _copy_merge_body(row_ref, idx_ref, in_ref, out_ref):
    out_ref[...] = in_ref[...]
    idx = idx_ref[0]

    @pl.when(pl.program_id(0) == idx // _BLOCK_R)
    def _splice_row():
        r = idx % _BLOCK_R
        out_ref[pl.ds(r, 1), :] = row_ref[...].reshape(1, _N_COLS)


def kernel(tensor, tensor_slice_index, probability):
    idx = jnp.asarray(tensor_slice_index, jnp.int32).reshape((1,))
    prob16 = jnp.full((_LANES,), probability, jnp.float32)
    u = jnp.asarray(_U_ROW)
    row_tile = _sc_flip_row(idx, prob16, u, tensor)
    grid = _N_ROWS // _BLOCK_R
    out = pl.pallas_call(
        _copy_merge_body,
        grid=(grid,),
        in_specs=[
            pl.BlockSpec((_N_COLS,), lambda i: (0,)),
            pl.BlockSpec(memory_space=pltpu.SMEM),
            pl.BlockSpec((_BLOCK_R, _N_COLS), lambda i: (i, 0)),
        ],
        out_specs=pl.BlockSpec((_BLOCK_R, _N_COLS), lambda i: (i, 0)),
        out_shape=jax.ShapeDtypeStruct((_N_ROWS, _N_COLS), jnp.float32),
    )(row_tile, idx, tensor)
    return (out, tensor_slice_index)


# speedup vs baseline: 38.6042x; 1.0186x over previous
import functools

import numpy as np
import jax
import jax.numpy as jnp
from jax import lax
from jax.experimental import pallas as pl
from jax.experimental.pallas import tpu as pltpu
from jax.experimental.pallas import tpu_sc as plsc

_N_ROWS = 16384
_N_COLS = 2048
_BLOCK_R = 1024
_N_WORKERS = 32
_COLS_PER_W = _N_COLS // _N_WORKERS
_LANES = 16


def _uniform_consts() -> np.ndarray:
    ks0, ks1 = np.uint32(0), np.uint32(42)
    ks2 = np.uint32(ks0 ^ ks1 ^ np.uint32(0x1BD11BDA))
    ks = [ks0, ks1, ks2]
    rot = [(13, 15, 26, 6), (17, 29, 16, 24)]

    def rotl(x, r):
        r = np.uint32(r)
        return ((x << r) | (x >> np.uint32(32 - r))).astype(np.uint32)

    x0 = np.full(_N_COLS, ks0, dtype=np.uint32)
    x1 = (np.arange(_N_COLS, dtype=np.uint32) + ks1).astype(np.uint32)
    for i in range(5):
        for r in rot[i % 2]:
            x0 = (x0 + x1).astype(np.uint32)
            x1 = rotl(x1, r)
            x1 = (x1 ^ x0).astype(np.uint32)
        x0 = (x0 + ks[(i + 1) % 3]).astype(np.uint32)
        x1 = (x1 + ks[(i + 2) % 3] + np.uint32(i + 1)).astype(np.uint32)
    bits = (x0 ^ x1).astype(np.uint32)
    fb = ((bits >> np.uint32(9)) | np.uint32(0x3F800000)).astype(np.uint32)
    u = fb.view(np.float32) - np.float32(1.0)
    return u


_U_ROW = _uniform_consts()


def _sc_flip_body(idx_hbm, prob_hbm, u_hbm, tensor_hbm, out_hbm,
                  idx_v, row_v, u_v, prob_v, res_v, sem):
    w = lax.axis_index("s") * 2 + lax.axis_index("c")
    base = w * _COLS_PER_W
    pltpu.sync_copy(idx_hbm, idx_v)
    pltpu.async_copy(tensor_hbm.at[idx_v], row_v, sem).wait()
    pltpu.sync_copy(u_hbm.at[pl.ds(base, _COLS_PER_W)], u_v)
    pltpu.sync_copy(prob_hbm, prob_v)
    pv = prob_v[...]
    one = jnp.full((_LANES,), 1.0, jnp.float32)
    zero = jnp.full((_LANES,), 0.0, jnp.float32)
    for j in range(_COLS_PER_W // _LANES):
        u16 = u_v[pl.ds(j * _LANES, _LANES)]
        r16 = row_v[0, pl.ds(base + j * _LANES, _LANES)]
        s16 = jnp.where(u16 < pv, one, zero)
        res_v[pl.ds(j * _LANES, _LANES)] = jnp.abs(r16 - s16)
    pltpu.sync_copy(res_v, out_hbm.at[pl.ds(base, _COLS_PER_W)])


def _sc_flip_row(idx, prob16, u, tensor):
    mesh = plsc.VectorSubcoreMesh(core_axis_name="c", subcore_axis_name="s")
    run = functools.partial(
        pl.kernel,
        out_type=jax.ShapeDtypeStruct((_N_COLS,), jnp.float32),
        mesh=mesh,
        scratch_types=[
            pltpu.VMEM((1,), jnp.int32),
            pltpu.VMEM((1, _N_COLS), jnp.float32),
            pltpu.VMEM((_COLS_PER_W,), jnp.float32),
            pltpu.VMEM((_LANES,), jnp.float32),
            pltpu.VMEM((_COLS_PER_W,), jnp.float32),
            pltpu.SemaphoreType.DMA,
        ],
    )(_sc_flip_body)
    return run(idx, prob16, u, tensor)


def _copy_body(in_ref, out_ref):
    out_ref[...] = in_ref[...]


def _splice_body(idx_ref, row_hbm, big_hbm, out_hbm, sem):
    cp = pltpu.make_async_copy(row_hbm, out_hbm.at[idx_ref[0]], sem)
    cp.start()
    cp.wait()


def kernel(tensor, tensor_slice_index, probability):
    idx = jnp.asarray(tensor_slice_index, jnp.int32).reshape((1,))
    prob16 = jnp.full((_LANES,), probability, jnp.float32)
    u = jnp.asarray(_U_ROW)
    row = _sc_flip_row(idx, prob16, u, tensor)
    grid = _N_ROWS // _BLOCK_R
    copied = pl.pallas_call(
        _copy_body,
        grid=(grid,),
        in_specs=[pl.BlockSpec((_BLOCK_R, _N_COLS), lambda i: (i, 0))],
        out_specs=pl.BlockSpec((_BLOCK_R, _N_COLS), lambda i: (i, 0)),
        out_shape=jax.ShapeDtypeStruct((_N_ROWS, _N_COLS), jnp.float32),
    )(tensor)
    out = pl.pallas_call(
        _splice_body,
        in_specs=[
            pl.BlockSpec(memory_space=pltpu.SMEM),
            pl.BlockSpec(memory_space=pltpu.MemorySpace.HBM),
            pl.BlockSpec(memory_space=pltpu.MemorySpace.HBM),
        ],
        out_specs=pl.BlockSpec(memory_space=pltpu.MemorySpace.HBM),
        out_shape=jax.ShapeDtypeStruct((_N_ROWS, _N_COLS), jnp.float32),
        scratch_shapes=[pltpu.SemaphoreType.DMA],
        input_output_aliases={2: 0},
    )(idx, row, copied)
    return (out, tensor_slice_index)


# speedup vs baseline: 47.4927x; 1.2302x over previous
import numpy as np
import jax
import jax.numpy as jnp
from jax.experimental import pallas as pl
from jax.experimental.pallas import tpu as pltpu

_N_ROWS = 16384
_N_COLS = 2048
_CHUNK_R = 1024
_N_CHUNKS = _N_ROWS // _CHUNK_R
_N_BUF = 4


def _uniform_consts() -> np.ndarray:
    ks0, ks1 = np.uint32(0), np.uint32(42)
    ks2 = np.uint32(ks0 ^ ks1 ^ np.uint32(0x1BD11BDA))
    ks = [ks0, ks1, ks2]
    rot = [(13, 15, 26, 6), (17, 29, 16, 24)]

    def rotl(x, r):
        r = np.uint32(r)
        return ((x << r) | (x >> np.uint32(32 - r))).astype(np.uint32)

    x0 = np.full(_N_COLS, ks0, dtype=np.uint32)
    x1 = (np.arange(_N_COLS, dtype=np.uint32) + ks1).astype(np.uint32)
    for i in range(5):
        for r in rot[i % 2]:
            x0 = (x0 + x1).astype(np.uint32)
            x1 = rotl(x1, r)
            x1 = (x1 ^ x0).astype(np.uint32)
        x0 = (x0 + ks[(i + 1) % 3]).astype(np.uint32)
        x1 = (x1 + ks[(i + 2) % 3] + np.uint32(i + 1)).astype(np.uint32)
    bits = (x0 ^ x1).astype(np.uint32)
    fb = ((bits >> np.uint32(9)) | np.uint32(0x3F800000)).astype(np.uint32)
    u = fb.view(np.float32) - np.float32(1.0)
    return np.broadcast_to(u, (8, _N_COLS)).copy()


_U_TILE = _uniform_consts()


def _ring_body(idx_ref, prob_ref, u_ref, in_hbm, out_hbm, buf, in_sems, out_sems):
    idx = idx_ref[0]

    def in_copy(c, b):
        return pltpu.make_async_copy(
            in_hbm.at[pl.ds(c * _CHUNK_R, _CHUNK_R), :], buf.at[b],
            in_sems.at[b])

    def out_copy(c, b):
        return pltpu.make_async_copy(
            buf.at[b], out_hbm.at[pl.ds(c * _CHUNK_R, _CHUNK_R), :],
            out_sems.at[b])

    for c in range(_N_BUF):
        in_copy(c, c).start()

    for c in range(_N_CHUNKS):
        b = c % _N_BUF
        in_copy(c, b).wait()

        @pl.when(c == idx // _CHUNK_R)
        def _flip_row():
            r = idx % _CHUNK_R
            row = buf[b, pl.ds(r, 1), :]
            sample = (u_ref[pl.ds(0, 1), :] < prob_ref[0]).astype(jnp.float32)
            buf[b, pl.ds(r, 1), :] = jnp.abs(row - sample)

        out_copy(c, b).start()
        nxt = c + _N_BUF
        if nxt < _N_CHUNKS:
            out_copy(c, b).wait()
            in_copy(nxt, b).start()
    for c in range(_N_CHUNKS - _N_BUF, _N_CHUNKS):
        out_copy(c, c % _N_BUF).wait()


def kernel(tensor, tensor_slice_index, probability):
    idx = jnp.asarray(tensor_slice_index, jnp.int32).reshape((1,))
    prob = jnp.asarray(probability, jnp.float32).reshape((1,))
    u = jnp.asarray(_U_TILE)
    out = pl.pallas_call(
        _ring_body,
        in_specs=[
            pl.BlockSpec(memory_space=pltpu.SMEM),
            pl.BlockSpec(memory_space=pltpu.SMEM),
            pl.BlockSpec(memory_space=pltpu.VMEM),
            pl.BlockSpec(memory_space=pltpu.MemorySpace.HBM),
        ],
        out_specs=pl.BlockSpec(memory_space=pltpu.MemorySpace.HBM),
        out_shape=jax.ShapeDtypeStruct((_N_ROWS, _N_COLS), jnp.float32),
        scratch_shapes=[
            pltpu.VMEM((_N_BUF, _CHUNK_R, _N_COLS), jnp.float32),
            pltpu.SemaphoreType.DMA((_N_BUF,)),
            pltpu.SemaphoreType.DMA((_N_BUF,)),
        ],
    )(idx, prob, u, tensor)
    return (out, tensor_slice_index)


# speedup vs baseline: 47.6666x; 1.0037x over previous
import numpy as np
import jax
import jax.numpy as jnp
from jax.experimental import pallas as pl
from jax.experimental.pallas import tpu as pltpu

_N_ROWS = 16384
_N_COLS = 2048
_CHUNK_R = 2048
_N_CHUNKS = _N_ROWS // _CHUNK_R
_N_BUF = 3


def _uniform_consts() -> np.ndarray:
    ks0, ks1 = np.uint32(0), np.uint32(42)
    ks2 = np.uint32(ks0 ^ ks1 ^ np.uint32(0x1BD11BDA))
    ks = [ks0, ks1, ks2]
    rot = [(13, 15, 26, 6), (17, 29, 16, 24)]

    def rotl(x, r):
        r = np.uint32(r)
        return ((x << r) | (x >> np.uint32(32 - r))).astype(np.uint32)

    x0 = np.full(_N_COLS, ks0, dtype=np.uint32)
    x1 = (np.arange(_N_COLS, dtype=np.uint32) + ks1).astype(np.uint32)
    for i in range(5):
        for r in rot[i % 2]:
            x0 = (x0 + x1).astype(np.uint32)
            x1 = rotl(x1, r)
            x1 = (x1 ^ x0).astype(np.uint32)
        x0 = (x0 + ks[(i + 1) % 3]).astype(np.uint32)
        x1 = (x1 + ks[(i + 2) % 3] + np.uint32(i + 1)).astype(np.uint32)
    bits = (x0 ^ x1).astype(np.uint32)
    fb = ((bits >> np.uint32(9)) | np.uint32(0x3F800000)).astype(np.uint32)
    u = fb.view(np.float32) - np.float32(1.0)
    return np.broadcast_to(u, (8, _N_COLS)).copy()


_U_TILE = _uniform_consts()


def _ring_body(idx_ref, prob_ref, u_ref, in_hbm, out_hbm, buf, in_sems, out_sems):
    idx = idx_ref[0]

    def in_copy(c, b):
        return pltpu.make_async_copy(
            in_hbm.at[pl.ds(c * _CHUNK_R, _CHUNK_R), :], buf.at[b],
            in_sems.at[b])

    def out_copy(c, b):
        return pltpu.make_async_copy(
            buf.at[b], out_hbm.at[pl.ds(c * _CHUNK_R, _CHUNK_R), :],
            out_sems.at[b])

    for c in range(_N_BUF):
        in_copy(c, c).start()

    for c in range(_N_CHUNKS):
        b = c % _N_BUF
        in_copy(c, b).wait()

        @pl.when(c == idx // _CHUNK_R)
        def _flip_row():
            r = idx % _CHUNK_R
            row = buf[b, pl.ds(r, 1), :]
            sample = (u_ref[pl.ds(0, 1), :] < prob_ref[0]).astype(jnp.float32)
            buf[b, pl.ds(r, 1), :] = jnp.abs(row - sample)

        out_copy(c, b).start()
        nxt = c + _N_BUF
        if nxt < _N_CHUNKS:
            out_copy(c, b).wait()
            in_copy(nxt, b).start()
    for c in range(_N_CHUNKS - _N_BUF, _N_CHUNKS):
        out_copy(c, c % _N_BUF).wait()


def kernel(tensor, tensor_slice_index, probability):
    idx = jnp.asarray(tensor_slice_index, jnp.int32).reshape((1,))
    prob = jnp.asarray(probability, jnp.float32).reshape((1,))
    u = jnp.asarray(_U_TILE)
    out = pl.pallas_call(
        _ring_body,
        in_specs=[
            pl.BlockSpec(memory_space=pltpu.SMEM),
            pl.BlockSpec(memory_space=pltpu.SMEM),
            pl.BlockSpec(memory_space=pltpu.VMEM),
            pl.BlockSpec(memory_space=pltpu.MemorySpace.HBM),
        ],
        out_specs=pl.BlockSpec(memory_space=pltpu.MemorySpace.HBM),
        out_shape=jax.ShapeDtypeStruct((_N_ROWS, _N_COLS), jnp.float32),
        scratch_shapes=[
            pltpu.VMEM((_N_BUF, _CHUNK_R, _N_COLS), jnp.float32),
            pltpu.SemaphoreType.DMA((_N_BUF,)),
            pltpu.SemaphoreType.DMA((_N_BUF,)),
        ],
    )(idx, prob, u, tensor)
    return (out, tensor_slice_index)
